# baseline (speedup 1.0000x reference)
import jax
import jax.numpy as jnp
from jax.experimental import pallas as pl
from jax.experimental.pallas import tpu as pltpu

_BM = 1024
_NBUF = 8


def _copy(x_hbm, xbuf, sems, block, slot):
    return pltpu.make_async_copy(
        x_hbm.at[pl.ds(block * _BM, _BM), :], xbuf.at[slot], sems.at[slot])


def _gate_matmul(x_hbm, wg_ref, o_hbm, xbuf, wgt, obuf, sems, osem):
    i = pl.program_id(0)
    nsteps = pl.num_programs(0)

    @pl.when(i == 0)
    def _warmup():
        for b in range(_NBUF):
            _copy(x_hbm, xbuf, sems, b, b).start()
        wgt[...] = wg_ref[...].T.astype(jnp.bfloat16)

    slot = jax.lax.rem(i, _NBUF)
    _copy(x_hbm, xbuf, sems, i, slot).wait()
    obuf[:, pl.ds(i * _BM, _BM)] = jnp.dot(
        xbuf[slot].astype(jnp.bfloat16), wgt[...],
        preferred_element_type=jnp.float32).T

    nxt = i + _NBUF

    @pl.when(nxt < nsteps)
    def _prefetch():
        _copy(x_hbm, xbuf, sems, nxt, slot).start()

    @pl.when(i == nsteps - 1)
    def _flush():
        out_dma = pltpu.make_async_copy(obuf, o_hbm, osem)
        out_dma.start()
        out_dma.wait()


def kernel(x, wg):
    m, k = x.shape
    e = wg.shape[0]
    out_t = pl.pallas_call(
        _gate_matmul,
        grid=(m // _BM,),
        in_specs=[
            pl.BlockSpec(memory_space=pl.ANY),
            pl.BlockSpec((e, k), lambda i: (0, 0)),
        ],
        out_specs=pl.BlockSpec(memory_space=pl.ANY),
        out_shape=jax.ShapeDtypeStruct((e, m), jnp.float32),
        scratch_shapes=[
            pltpu.VMEM((_NBUF, _BM, k), jnp.float32),
            pltpu.VMEM((k, e), jnp.bfloat16),
            pltpu.VMEM((e, m), jnp.float32),
            pltpu.SemaphoreType.DMA((_NBUF,)),
            pltpu.SemaphoreType.DMA,
        ],
    )(x, wg)
    return out_t.T


# speedup vs baseline: 1.0187x; 1.0187x over previous
import jax
import jax.numpy as jnp
from jax.experimental import pallas as pl
from jax.experimental.pallas import tpu as pltpu

_BM = 1024
_NBUF = 8
_OGRP = 4


def _copy(x_hbm, xbuf, sems, block, slot):
    return pltpu.make_async_copy(
        x_hbm.at[pl.ds(block * _BM, _BM), :], xbuf.at[slot], sems.at[slot])


def _gate_matmul(x_hbm, wg_ref, o_ref, xbuf, wgt, sems):
    i = pl.program_id(0)
    nsteps = pl.num_programs(0)

    @pl.when(i == 0)
    def _warmup():
        for b in range(_NBUF):
            _copy(x_hbm, xbuf, sems, b, b).start()
        wgt[...] = wg_ref[...].T.astype(jnp.bfloat16)

    slot = jax.lax.rem(i, _NBUF)
    _copy(x_hbm, xbuf, sems, i, slot).wait()
    o_ref[:, pl.ds(jax.lax.rem(i, _OGRP) * _BM, _BM)] = jnp.dot(
        xbuf[slot].astype(jnp.bfloat16), wgt[...],
        preferred_element_type=jnp.float32).T

    nxt = i + _NBUF

    @pl.when(nxt < nsteps)
    def _prefetch():
        _copy(x_hbm, xbuf, sems, nxt, slot).start()


def kernel(x, wg):
    m, k = x.shape
    e = wg.shape[0]
    out_t = pl.pallas_call(
        _gate_matmul,
        grid=(m // _BM,),
        in_specs=[
            pl.BlockSpec(memory_space=pl.ANY),
            pl.BlockSpec((e, k), lambda i: (0, 0)),
        ],
        out_specs=pl.BlockSpec((e, _OGRP * _BM), lambda i: (0, i // _OGRP)),
        out_shape=jax.ShapeDtypeStruct((e, m), jnp.float32),
        scratch_shapes=[
            pltpu.VMEM((_NBUF, _BM, k), jnp.float32),
            pltpu.VMEM((k, e), jnp.bfloat16),
            pltpu.SemaphoreType.DMA((_NBUF,)),
        ],
    )(x, wg)
    return out_t.T


# speedup vs baseline: 1.0306x; 1.0116x over previous
import jax
import jax.numpy as jnp
from jax.experimental import pallas as pl
from jax.experimental.pallas import tpu as pltpu

_BM = 1024
_NBUF = 8


def _copy(x_hbm, xbuf, sems, block, slot):
    return pltpu.make_async_copy(
        x_hbm.at[pl.ds(block * _BM, _BM), :], xbuf.at[slot], sems.at[slot])


def _gate_matmul(x_hbm, wg_ref, o_ref, xbuf, wgt, sems):
    i = pl.program_id(0)
    nsteps = pl.num_programs(0)

    @pl.when(i == 0)
    def _warmup():
        for b in range(_NBUF):
            _copy(x_hbm, xbuf, sems, b, b).start()
        wgt[...] = wg_ref[...].T.astype(jnp.bfloat16)

    slot = jax.lax.rem(i, _NBUF)
    _copy(x_hbm, xbuf, sems, i, slot).wait()
    o_ref[...] = jnp.dot(xbuf[slot].astype(jnp.bfloat16), wgt[...],
                         preferred_element_type=jnp.float32).T

    nxt = i + _NBUF

    @pl.when(nxt < nsteps)
    def _prefetch():
        _copy(x_hbm, xbuf, sems, nxt, slot).start()


def kernel(x, wg):
    m, k = x.shape
    e = wg.shape[0]
    out_t = pl.pallas_call(
        _gate_matmul,
        grid=(m // _BM,),
        in_specs=[
            pl.BlockSpec(memory_space=pl.ANY),
            pl.BlockSpec((e, k), lambda i: (0, 0)),
        ],
        out_specs=pl.BlockSpec((e, _BM), lambda i: (0, i)),
        out_shape=jax.ShapeDtypeStruct((e, m), jnp.float32),
        scratch_shapes=[
            pltpu.VMEM((_NBUF, _BM, k), jnp.float32),
            pltpu.VMEM((k, e), jnp.bfloat16),
            pltpu.SemaphoreType.DMA((_NBUF,)),
        ],
    )(x, wg)
    return out_t.T
